# baseline (speedup 1.0000x reference)
import jax
import jax.numpy as jnp
from jax import lax
from jax.experimental import pallas as pl
from jax.experimental.pallas import tpu as pltpu

_B = 64
_N = 128
_F = 8
_SD = 14
_K = 4
_G = 32


def _graph_body(ve_ref, nodes_ref, edges_ref, mem_ref):
    ve = ve_ref[:]
    nodes = nodes_ref[:]
    diff = nodes - ve
    d2 = jnp.sum(diff * diff, axis=2, keepdims=True)

    io_n1 = lax.broadcasted_iota(jnp.int32, (_G, _N, 1), 1)
    m = jnp.min(d2, axis=1, keepdims=True)
    closest = jnp.min(jnp.where(d2 == m, io_n1, _N),
                      axis=1, keepdims=True).astype(jnp.int32)

    cost = edges_ref[:, 0]
    for c in range(1, 6):
        cost = jnp.minimum(cost, edges_ref[:, c])

    sub3 = lax.broadcasted_iota(jnp.int32, (_G, _N, _N), 1)
    lane3 = lax.broadcasted_iota(jnp.int32, (_G, _N, _N), 2)
    lane_row = lax.broadcasted_iota(jnp.int32, (_G, 1, _N), 2)
    eye = sub3 == lane3
    inf = jnp.float32(jnp.inf)

    d_row = jnp.min(jnp.where(sub3 == closest, cost, inf),
                    axis=1, keepdims=True)
    d_row = jnp.where(lane_row == closest, jnp.float32(0.0), d_row)

    def bf_cond(carry):
        _, changed, it = carry
        return jnp.logical_and(changed, it < _N - 1)

    def bf_body(carry):
        d, _, it = carry
        d_col = jnp.min(jnp.where(eye, jnp.broadcast_to(d, (_G, _N, _N)), inf),
                        axis=2, keepdims=True)
        relaxed = jnp.min(d_col + cost, axis=1, keepdims=True)
        new_d = jnp.minimum(d, relaxed)
        return new_d, jnp.any(new_d < d), it + jnp.int32(1)

    d_row, _, _ = lax.while_loop(
        bf_cond, bf_body, (d_row, jnp.array(True), jnp.int32(0)))

    arows = []
    for c in range(6):
        ec = edges_ref[:, c]
        arows.append(jnp.sum(jnp.where(sub3 == closest, ec, 0.0),
                             axis=1, keepdims=True))

    nodes14 = jnp.concatenate(
        [nodes, jnp.zeros((_G, _N, _SD - _F), jnp.float32)], axis=2)
    node_rowio = lax.broadcasted_iota(jnp.int32, (_G, _N, _SD), 1)
    lane14 = lax.broadcasted_iota(jnp.int32, (_G, 1, _SD), 2)

    dw = d_row
    rows = []
    for k in range(_K):
        mk = jnp.min(dw, axis=2, keepdims=True)
        ik = jnp.min(jnp.where(dw == mk, lane_row, _N),
                     axis=2, keepdims=True).astype(jnp.int32)
        dw = jnp.where(lane_row == ik, inf, dw)
        row = jnp.sum(jnp.where(node_rowio == ik, nodes14, 0.0),
                      axis=1, keepdims=True)
        for c in range(6):
            val = jnp.sum(jnp.where(lane_row == ik, arows[c], 0.0),
                          axis=2, keepdims=True)
            row = row + jnp.where(lane14 == _F + c, val, 0.0)
        rows.append(row)
    mem_ref[:] = jnp.concatenate(rows, axis=1)


def _graph_call(vision_enc, nodes, edges_t):
    return pl.pallas_call(
        _graph_body,
        grid=(_B // _G,),
        in_specs=[
            pl.BlockSpec((_G, 1, _F), lambda b: (b, 0, 0)),
            pl.BlockSpec((_G, _N, _F), lambda b: (b, 0, 0)),
            pl.BlockSpec((_G, 6, _N, _N), lambda b: (b, 0, 0, 0)),
        ],
        out_specs=pl.BlockSpec((_G, _K, _SD), lambda b: (b, 0, 0)),
        out_shape=jax.ShapeDtypeStruct((_B, _K, _SD), jnp.float32),
        compiler_params=pltpu.CompilerParams(
            dimension_semantics=("arbitrary",)),
    )(vision_enc, nodes, edges_t)


def _tail_body(tcol_ref, mem_ref,
               t1w_ref, t1b_ref, t2w_ref, t2b_ref,
               wq_ref, bq_ref, wk_ref, bk_ref, wv_ref, bv_ref,
               wo_ref, bo_ref, ln1g_ref, ln1b_ref,
               f1w_ref, f1b_ref, f2w_ref, f2b_ref,
               ln2g_ref, ln2b_ref,
               h1w_ref, h1b_ref, h2w_ref, h2b_ref, h3w_ref, h3b_ref,
               out_ref):
    tcol = tcol_ref[:]
    t = jnp.maximum(tcol @ t1w_ref[:] + t1b_ref[:], 0.0)
    te = t @ t2w_ref[:] + t2b_ref[:]
    mem = mem_ref[:]
    s = jnp.concatenate([te, mem], axis=0)

    q = s @ wq_ref[:] + bq_ref[:]
    k = s @ wk_ref[:] + bk_ref[:]
    v = s @ wv_ref[:] + bv_ref[:]
    scores = lax.dot_general(q, k, (((1,), (1,)), ((), ())))
    scores = scores / jnp.sqrt(jnp.float32(_SD))

    rio = lax.broadcasted_iota(jnp.int32, (5 * _B, 1), 0)
    cio = lax.broadcasted_iota(jnp.int32, (1, 5 * _B), 1)
    g_r = jnp.where(rio < _B, rio, (rio - _B) // 4)
    g_c = jnp.where(cio < _B, cio, (cio - _B) // 4)
    mask = g_r == g_c
    neg = jnp.float32(-jnp.inf)
    scores = jnp.where(mask, scores, neg)
    mx = jnp.max(scores, axis=1, keepdims=True)
    e = jnp.exp(scores - mx)
    attn_w = e / jnp.sum(e, axis=1, keepdims=True)
    att = attn_w @ v
    att = att @ wo_ref[:] + bo_ref[:]

    def ln(x, g, b):
        mu = jnp.mean(x, axis=1, keepdims=True)
        var = jnp.mean((x - mu) ** 2, axis=1, keepdims=True)
        return (x - mu) / jnp.sqrt(var + 1e-5) * g + b

    s1 = ln(s + att, ln1g_ref[:], ln1b_ref[:])
    ff = jnp.maximum(s1 @ f1w_ref[:] + f1b_ref[:], 0.0)
    ff = ff @ f2w_ref[:] + f2b_ref[:]
    s2 = ln(s1 + ff, ln2g_ref[:], ln2b_ref[:])

    t_final = s2[0:_B, :]
    m_final = s2[_B:, :]
    prow = lax.broadcasted_iota(jnp.int32, (_B, 4 * _B), 0)
    pcol = lax.broadcasted_iota(jnp.int32, (_B, 4 * _B), 1)
    pmat = jnp.where(prow == pcol // 4, jnp.float32(0.25), jnp.float32(0.0))
    m_mean = pmat @ m_final
    pooled = jnp.concatenate([t_final, m_mean], axis=1)

    h = jnp.maximum(pooled @ h1w_ref[:] + h1b_ref[:], 0.0)
    h = jnp.maximum(h @ h2w_ref[:] + h2b_ref[:], 0.0)
    out_ref[:] = h @ h3w_ref[:] + h3b_ref[:]


def _tail_call(tcol, mem2d, p):
    def t2(name):
        return p[name].T
    def b2(name):
        return p[name][None, :]
    operands = [
        tcol, mem2d,
        t2('tenc1_w'), b2('tenc1_b'), t2('tenc2_w'), b2('tenc2_b'),
        t2('wq'), b2('bq'), t2('wk'), b2('bk'), t2('wv'), b2('bv'),
        t2('wo'), b2('bo'), b2('ln1_g'), b2('ln1_b'),
        t2('ff1_w'), b2('ff1_b'), t2('ff2_w'), b2('ff2_b'),
        b2('ln2_g'), b2('ln2_b'),
        t2('h1_w'), b2('h1_b'), t2('h2_w'), b2('h2_b'),
        t2('h3_w'), b2('h3_b'),
    ]
    return pl.pallas_call(
        _tail_body,
        out_shape=jax.ShapeDtypeStruct((_B, 6), jnp.float32),
    )(*operands)


def _conv2d(x, w, b, padding):
    out = lax.conv_general_dilated(x, w, window_strides=(1, 1), padding=padding,
                                   dimension_numbers=('NCHW', 'OIHW', 'NCHW'))
    return out + b[None, :, None, None]


def _avgpool2(x):
    s = lax.reduce_window(x, 0.0, lax.add, (1, 1, 2, 2), (1, 1, 2, 2), 'VALID')
    return s / 4.0


def kernel(x, nodes, edges, params):
    p = params
    tcol = x[:, :, 0, 0]
    xv = x - 0.5
    h = jax.nn.relu(_avgpool2(_conv2d(xv, p['conv1_w'], p['conv1_b'], 'VALID')))
    h = jax.nn.relu(_avgpool2(_conv2d(h, p['conv2_w'], p['conv2_b'], 'SAME')))
    h = jax.nn.relu(_conv2d(h, p['conv3_w'], p['conv3_b'], 'SAME'))
    h = h.reshape(_B, -1)
    vision_enc = h @ p['venc_w'].T + p['venc_b']

    edges_t = jnp.moveaxis(edges, -1, 1)
    mem_seq = _graph_call(vision_enc[:, None, :], nodes, edges_t)
    mem2d = mem_seq.reshape(_B * _K, _SD)
    return _tail_call(tcol, mem2d, params)
